# baseline (speedup 1.0000x reference)
import functools

import jax
import jax.numpy as jnp
from jax import lax
from jax.experimental import pallas as pl
from jax.experimental.pallas import tpu as pltpu
from jax.experimental.pallas import tpu_sc as plsc

MAP_H = 64
MAP_W = 128
DIM = 256
N_ROWS = MAP_H * MAP_W
N_WORKERS = 32
ROWS_PER_WORKER = N_ROWS // N_WORKERS
N_GROUPS = ROWS_PER_WORKER // 16
LANES = 16


def _sc_body(x_hbm, w_hbm, val_hbm, idx_hbm, x_v, slab_v, val_v, idx_v):
    c = lax.axis_index("c")
    s = lax.axis_index("s")
    wid = s * 2 + c
    base = wid * ROWS_PER_WORKER

    pltpu.sync_copy(x_hbm, x_v)
    pltpu.sync_copy(w_hbm.at[pl.ds(base, ROWS_PER_WORKER)], slab_v)

    lane = lax.iota(jnp.int32, LANES)

    def chunk_step(k, accs):
        accs = list(accs)
        xv = x_v[pl.ds(k * LANES, LANES)]
        for j in range(LANES):
            d = k * LANES + j
            xs = jnp.full((LANES,), xv[j], dtype=jnp.float32)
            col = jnp.full((LANES,), d, dtype=jnp.int32)
            for g in range(N_GROUPS):
                row = lane + (g * LANES)
                wv = plsc.load_gather(slab_v, [row, col])
                diff = wv - xs
                accs[g] = accs[g] + diff * diff
        return tuple(accs)

    init = tuple(jnp.zeros((LANES,), jnp.float32) for _ in range(N_GROUPS))
    accs = lax.fori_loop(0, DIM // LANES, chunk_step, init)

    best = accs[0]
    brow = base + lane
    for g in range(1, N_GROUPS):
        rowv = base + g * LANES + lane
        m = accs[g] < best
        best = jnp.where(m, accs[g], best)
        brow = jnp.where(m, rowv, brow)

    val_v[...] = best
    idx_v[...] = brow
    pltpu.sync_copy(val_v, val_hbm.at[wid])
    pltpu.sync_copy(idx_v, idx_hbm.at[wid])


@functools.partial(
    pl.kernel,
    out_type=(
        jax.ShapeDtypeStruct((N_WORKERS, LANES), jnp.float32),
        jax.ShapeDtypeStruct((N_WORKERS, LANES), jnp.int32),
    ),
    mesh=plsc.VectorSubcoreMesh(core_axis_name="c", subcore_axis_name="s"),
    compiler_params=pltpu.CompilerParams(
        use_tc_tiling_on_sc=False, needs_layout_passes=False
    ),
    scratch_types=(
        pltpu.VMEM((DIM,), jnp.float32),
        pltpu.VMEM((ROWS_PER_WORKER, DIM), jnp.float32),
        pltpu.VMEM((LANES,), jnp.float32),
        pltpu.VMEM((LANES,), jnp.int32),
    ),
)
def _sc_candidates(x_hbm, w_hbm, val_hbm, idx_hbm, x_v, slab_v, val_v, idx_v):
    _sc_body(x_hbm, w_hbm, val_hbm, idx_hbm, x_v, slab_v, val_v, idx_v)


def _tc_finish_body(val_ref, idx_ref, dist_ref, map_ref):
    v = val_ref[...]
    r = idx_ref[...]
    mn = jnp.min(v)
    cand = jnp.where(v == mn, r, jnp.int32(N_ROWS))
    rmin = jnp.min(cand)
    dist_ref[0] = jnp.sqrt(jnp.maximum(mn, 0.0))
    map_ref[0] = rmin // MAP_W
    map_ref[1] = rmin % MAP_W


def _tc_finish(vals, idxs):
    return pl.pallas_call(
        _tc_finish_body,
        out_shape=(
            jax.ShapeDtypeStruct((1,), jnp.float32),
            jax.ShapeDtypeStruct((2,), jnp.int32),
        ),
        in_specs=[
            pl.BlockSpec(memory_space=pltpu.VMEM),
            pl.BlockSpec(memory_space=pltpu.VMEM),
        ],
        out_specs=(
            pl.BlockSpec(memory_space=pltpu.SMEM),
            pl.BlockSpec(memory_space=pltpu.SMEM),
        ),
    )(vals, idxs)


@jax.jit
def kernel(x, weights):
    wflat = weights.reshape(N_ROWS, DIM)
    vals, idxs = _sc_candidates(x, wflat)
    dist, mapidx = _tc_finish(vals, idxs)
    return dist[0], mapidx.astype(jnp.int64)


# speedup vs baseline: 1.8330x; 1.8330x over previous
import functools

import jax
import jax.numpy as jnp
from jax import lax
from jax.experimental import pallas as pl
from jax.experimental.pallas import tpu as pltpu
from jax.experimental.pallas import tpu_sc as plsc

MAP_H = 64
MAP_W = 128
DIM = 256
N_ROWS = MAP_H * MAP_W
N_WORKERS = 32
ROWS_PER_WORKER = N_ROWS // N_WORKERS
N_GROUPS = ROWS_PER_WORKER // 16
LANES = 16


def _sc_body(x_hbm, w_hbm, val_hbm, idx_hbm, x_v, slab_v, val_v, idx_v):
    c = lax.axis_index("c")
    s = lax.axis_index("s")
    wid = s * 2 + c
    base = wid * ROWS_PER_WORKER

    pltpu.sync_copy(x_hbm, x_v)
    pltpu.sync_copy(w_hbm.at[pl.ds(base, ROWS_PER_WORKER)], slab_v)

    lane = lax.iota(jnp.int32, LANES)
    n_chunks = DIM // LANES
    xk = [x_v[pl.ds(k * LANES, LANES)] for k in range(n_chunks)]

    def group_step(g, carry):
        best, brow = carry
        row0 = g * LANES
        sums = jnp.zeros((LANES,), jnp.float32)
        for r in range(LANES):
            a = jnp.zeros((LANES,), jnp.float32)
            b = jnp.zeros((LANES,), jnp.float32)
            for k in range(n_chunks):
                wv = slab_v[row0 + r, pl.ds(k * LANES, LANES)]
                diff = wv - xk[k]
                if k % 2 == 0:
                    a = a + diff * diff
                else:
                    b = b + diff * diff
            s_row = jnp.sum(a + b)
            sums = jnp.where(lane == r, s_row, sums)
        rowv = base + row0 + lane
        m = sums < best
        best = jnp.where(m, sums, best)
        brow = jnp.where(m, rowv, brow)
        return best, brow

    init = (jnp.full((LANES,), jnp.inf, jnp.float32), base + lane)
    best, brow = lax.fori_loop(0, N_GROUPS, group_step, init)

    val_v[...] = best
    idx_v[...] = brow
    pltpu.sync_copy(val_v, val_hbm.at[wid])
    pltpu.sync_copy(idx_v, idx_hbm.at[wid])


@functools.partial(
    pl.kernel,
    out_type=(
        jax.ShapeDtypeStruct((N_WORKERS, LANES), jnp.float32),
        jax.ShapeDtypeStruct((N_WORKERS, LANES), jnp.int32),
    ),
    mesh=plsc.VectorSubcoreMesh(core_axis_name="c", subcore_axis_name="s"),
    compiler_params=pltpu.CompilerParams(
        use_tc_tiling_on_sc=False, needs_layout_passes=False
    ),
    scratch_types=(
        pltpu.VMEM((DIM,), jnp.float32),
        pltpu.VMEM((ROWS_PER_WORKER, DIM), jnp.float32),
        pltpu.VMEM((LANES,), jnp.float32),
        pltpu.VMEM((LANES,), jnp.int32),
    ),
)
def _sc_candidates(x_hbm, w_hbm, val_hbm, idx_hbm, x_v, slab_v, val_v, idx_v):
    _sc_body(x_hbm, w_hbm, val_hbm, idx_hbm, x_v, slab_v, val_v, idx_v)


def _tc_finish_body(val_ref, idx_ref, dist_ref, map_ref):
    v = val_ref[...]
    r = idx_ref[...]
    mn = jnp.min(v)
    cand = jnp.where(v == mn, r, jnp.int32(N_ROWS))
    rmin = jnp.min(cand)
    dist_ref[0] = jnp.sqrt(jnp.maximum(mn, 0.0))
    map_ref[0] = rmin // MAP_W
    map_ref[1] = rmin % MAP_W


def _tc_finish(vals, idxs):
    return pl.pallas_call(
        _tc_finish_body,
        out_shape=(
            jax.ShapeDtypeStruct((1,), jnp.float32),
            jax.ShapeDtypeStruct((2,), jnp.int32),
        ),
        in_specs=[
            pl.BlockSpec(memory_space=pltpu.VMEM),
            pl.BlockSpec(memory_space=pltpu.VMEM),
        ],
        out_specs=(
            pl.BlockSpec(memory_space=pltpu.SMEM),
            pl.BlockSpec(memory_space=pltpu.SMEM),
        ),
    )(vals, idxs)


@jax.jit
def kernel(x, weights):
    wflat = weights.reshape(N_ROWS, DIM)
    vals, idxs = _sc_candidates(x, wflat)
    dist, mapidx = _tc_finish(vals, idxs)
    return dist[0], mapidx.astype(jnp.int64)


# speedup vs baseline: 2.6307x; 1.4352x over previous
import functools

import jax
import jax.numpy as jnp
from jax import lax
from jax.experimental import pallas as pl
from jax.experimental.pallas import tpu as pltpu
from jax.experimental.pallas import tpu_sc as plsc

MAP_H = 64
MAP_W = 128
DIM = 256
N_ROWS = MAP_H * MAP_W
N_WORKERS = 32
ROWS_PER_WORKER = N_ROWS // N_WORKERS
N_GROUPS = ROWS_PER_WORKER // 16
LANES = 16


def _sc_body(x_hbm, w_hbm, val_hbm, idx_hbm, x_v, slab_v, val_v, idx_v):
    c = lax.axis_index("c")
    s = lax.axis_index("s")
    wid = s * 2 + c
    base = wid * ROWS_PER_WORKER

    pltpu.sync_copy(x_hbm, x_v)
    pltpu.sync_copy(w_hbm.at[pl.ds(base, ROWS_PER_WORKER)], slab_v)

    lane = lax.iota(jnp.int32, LANES)
    n_chunks = DIM // LANES
    xk = [x_v[pl.ds(k * LANES, LANES)] for k in range(n_chunks)]

    def group_step(g, carry):
        best, brow = carry
        row0 = g * LANES
        sums = jnp.zeros((LANES,), jnp.float32)
        for r in range(LANES):
            a = jnp.zeros((LANES,), jnp.float32)
            b = jnp.zeros((LANES,), jnp.float32)
            for k in range(n_chunks):
                wv = slab_v[row0 + r, pl.ds(k * LANES, LANES)]
                diff = wv - xk[k]
                if k % 2 == 0:
                    a = a + diff * diff
                else:
                    b = b + diff * diff
            s_row = jnp.sum(a + b)
            sums = jnp.where(lane == r, s_row, sums)
        rowv = base + row0 + lane
        m = sums < best
        best = jnp.where(m, sums, best)
        brow = jnp.where(m, rowv, brow)
        return best, brow

    init = (jnp.full((LANES,), jnp.inf, jnp.float32), base + lane)
    best, brow = lax.fori_loop(0, N_GROUPS, group_step, init)

    val_v[...] = best
    idx_v[...] = brow
    pltpu.sync_copy(val_v, val_hbm.at[wid])
    pltpu.sync_copy(idx_v, idx_hbm.at[wid])


@functools.partial(
    pl.kernel,
    out_type=(
        jax.ShapeDtypeStruct((N_WORKERS, LANES), jnp.float32),
        jax.ShapeDtypeStruct((N_WORKERS, LANES), jnp.int32),
    ),
    mesh=plsc.VectorSubcoreMesh(core_axis_name="c", subcore_axis_name="s"),
    compiler_params=pltpu.CompilerParams(
        use_tc_tiling_on_sc=True, needs_layout_passes=False
    ),
    scratch_types=(
        pltpu.VMEM((DIM,), jnp.float32),
        pltpu.VMEM((ROWS_PER_WORKER, DIM), jnp.float32),
        pltpu.VMEM((LANES,), jnp.float32),
        pltpu.VMEM((LANES,), jnp.int32),
    ),
)
def _sc_candidates(x_hbm, w_hbm, val_hbm, idx_hbm, x_v, slab_v, val_v, idx_v):
    _sc_body(x_hbm, w_hbm, val_hbm, idx_hbm, x_v, slab_v, val_v, idx_v)


def _tc_finish_body(val_ref, idx_ref, dist_ref, map_ref):
    v = val_ref[...]
    r = idx_ref[...]
    mn = jnp.min(v)
    cand = jnp.where(v == mn, r, jnp.int32(N_ROWS))
    rmin = jnp.min(cand)
    dist_ref[0] = jnp.sqrt(jnp.maximum(mn, 0.0))
    map_ref[0] = rmin // MAP_W
    map_ref[1] = rmin % MAP_W


def _tc_finish(vals, idxs):
    return pl.pallas_call(
        _tc_finish_body,
        out_shape=(
            jax.ShapeDtypeStruct((1,), jnp.float32),
            jax.ShapeDtypeStruct((2,), jnp.int32),
        ),
        in_specs=[
            pl.BlockSpec(memory_space=pltpu.VMEM),
            pl.BlockSpec(memory_space=pltpu.VMEM),
        ],
        out_specs=(
            pl.BlockSpec(memory_space=pltpu.SMEM),
            pl.BlockSpec(memory_space=pltpu.SMEM),
        ),
    )(vals, idxs)


@jax.jit
def kernel(x, weights):
    wflat = weights.reshape(N_ROWS, DIM)
    vals, idxs = _sc_candidates(x, wflat)
    dist, mapidx = _tc_finish(vals, idxs)
    return dist[0], mapidx.astype(jnp.int64)


# speedup vs baseline: 2.6505x; 1.0075x over previous
import functools

import jax
import jax.numpy as jnp
from jax import lax
from jax.experimental import pallas as pl
from jax.experimental.pallas import tpu as pltpu
from jax.experimental.pallas import tpu_sc as plsc

MAP_H = 64
MAP_W = 128
DIM = 256
N_ROWS = MAP_H * MAP_W
N_WORKERS = 32
ROWS_PER_WORKER = N_ROWS // N_WORKERS
N_GROUPS = ROWS_PER_WORKER // 16
LANES = 16


def _sc_body(x_hbm, w_hbm, val_hbm, idx_hbm, x_v, slab_v, val_v, idx_v):
    c = lax.axis_index("c")
    s = lax.axis_index("s")
    wid = s * 2 + c
    base = wid * ROWS_PER_WORKER

    pltpu.sync_copy(x_hbm, x_v)
    pltpu.sync_copy(w_hbm.at[pl.ds(base, ROWS_PER_WORKER)], slab_v)

    lane = lax.iota(jnp.int32, LANES)
    n_chunks = DIM // LANES
    xk = [x_v[pl.ds(k * LANES, LANES)] for k in range(n_chunks)]

    def group_step(g, carry):
        best, brow = carry
        row0 = g * LANES
        sums = jnp.zeros((LANES,), jnp.float32)
        for r in range(LANES):
            a = jnp.zeros((LANES,), jnp.float32)
            b = jnp.zeros((LANES,), jnp.float32)
            for k in range(n_chunks):
                wv = slab_v[row0 + r, pl.ds(k * LANES, LANES)]
                diff = wv - xk[k]
                if k % 2 == 0:
                    a = a + diff * diff
                else:
                    b = b + diff * diff
            s_row = jnp.sum(a + b)
            sums = jnp.where(lane == r, s_row, sums)
        rowv = base + row0 + lane
        m = sums < best
        best = jnp.where(m, sums, best)
        brow = jnp.where(m, rowv, brow)
        return best, brow

    init = (jnp.full((LANES,), jnp.inf, jnp.float32), base + lane)
    best, brow = lax.fori_loop(0, N_GROUPS, group_step, init)

    val_v[...] = best
    idx_v[...] = brow
    pltpu.sync_copy(val_v, val_hbm.at[wid])
    pltpu.sync_copy(idx_v, idx_hbm.at[wid])


@functools.partial(
    pl.kernel,
    out_type=(
        jax.ShapeDtypeStruct((N_WORKERS, LANES), jnp.float32),
        jax.ShapeDtypeStruct((N_WORKERS, LANES), jnp.int32),
    ),
    mesh=plsc.VectorSubcoreMesh(core_axis_name="c", subcore_axis_name="s"),
    compiler_params=pltpu.CompilerParams(
        use_tc_tiling_on_sc=True,
        needs_layout_passes=False,
        skip_device_barrier=True,
    ),
    scratch_types=(
        pltpu.VMEM((DIM,), jnp.float32),
        pltpu.VMEM((ROWS_PER_WORKER, DIM), jnp.float32),
        pltpu.VMEM((LANES,), jnp.float32),
        pltpu.VMEM((LANES,), jnp.int32),
    ),
)
def _sc_candidates(x_hbm, w_hbm, val_hbm, idx_hbm, x_v, slab_v, val_v, idx_v):
    _sc_body(x_hbm, w_hbm, val_hbm, idx_hbm, x_v, slab_v, val_v, idx_v)


def _tc_finish_body(val_ref, idx_ref, dist_ref, map_ref):
    v = val_ref[...]
    r = idx_ref[...]
    mn = jnp.min(v)
    cand = jnp.where(v == mn, r, jnp.int32(N_ROWS))
    rmin = jnp.min(cand)
    dist_ref[0] = jnp.sqrt(jnp.maximum(mn, 0.0))
    map_ref[0] = rmin // MAP_W
    map_ref[1] = rmin % MAP_W


def _tc_finish(vals, idxs):
    return pl.pallas_call(
        _tc_finish_body,
        out_shape=(
            jax.ShapeDtypeStruct((1,), jnp.float32),
            jax.ShapeDtypeStruct((2,), jnp.int32),
        ),
        in_specs=[
            pl.BlockSpec(memory_space=pltpu.VMEM),
            pl.BlockSpec(memory_space=pltpu.VMEM),
        ],
        out_specs=(
            pl.BlockSpec(memory_space=pltpu.SMEM),
            pl.BlockSpec(memory_space=pltpu.SMEM),
        ),
    )(vals, idxs)


@jax.jit
def kernel(x, weights):
    wflat = weights.reshape(N_ROWS, DIM)
    vals, idxs = _sc_candidates(x, wflat)
    dist, mapidx = _tc_finish(vals, idxs)
    return dist[0], mapidx.astype(jnp.int64)
